# Initial kernel scaffold; baseline (speedup 1.0000x reference)
import jax
import jax.numpy as jnp
from jax.experimental import pallas as pl


def kernel(x_t, x_h, y_h, feature_weights):
    raise NotImplementedError("write your pallas kernel here")


# speedup vs baseline: 4.3710x; 4.3710x over previous
import functools

import jax
import jax.numpy as jnp
from jax.experimental import pallas as pl
from jax.experimental.pallas import tpu as pltpu

N_ANALOGS = 50
TW = 2
WS = 2 * TW + 1
NF = 8
N_CUR = 60
N_HIST = 16380
H = 16384
HPAD = H + 128


def _anen_body(xt_ref, xh_ref, y_ref, fw_ref, out_ref, d_ref):
    xh = xh_ref[:, :H]
    mean = jnp.sum(xh, axis=1, keepdims=True) / H
    var = jnp.sum((xh - mean) ** 2, axis=1, keepdims=True) / H
    std = jnp.maximum(jnp.sqrt(var), 1e-8)
    wn = fw_ref[:, :] / std

    C = 4096
    inf = jnp.float32(jnp.inf)
    for c in range(H // C):
        dis = jnp.zeros((N_CUR, C), dtype=jnp.float32)
        for f in range(NF):
            acc = jnp.zeros((N_CUR, C), dtype=jnp.float32)
            for t in range(WS):
                a = xt_ref[t:t + N_CUR, f:f + 1]
                b = xh_ref[f:f + 1, c * C + t:c * C + t + C]
                d = a - b
                acc = acc + d * d
            dis = dis + wn[f:f + 1, 0:1] * jnp.sqrt(acc)
        lane_c = jax.lax.broadcasted_iota(jnp.int32, (N_CUR, C), 1) + c * C
        d_ref[:, c * C:(c + 1) * C] = jnp.where(lane_c >= N_HIST, inf, dis)

    lane = jax.lax.broadcasted_iota(jnp.int32, (N_CUR, H), 1)

    yrow = y_ref[0:1, :]
    big = jnp.int32(1 << 30)
    lane_out = jax.lax.broadcasted_iota(jnp.int32, (N_CUR, N_ANALOGS), 1)

    def step(k, acc):
        dcur = d_ref[...]
        m = jnp.min(dcur, axis=1, keepdims=True)
        eq = dcur == m
        idx = jnp.min(jnp.where(eq, lane, big), axis=1, keepdims=True)
        onehot = lane == idx
        val = jnp.sum(jnp.where(onehot, yrow, 0.0), axis=1, keepdims=True)
        d_ref[...] = jnp.where(onehot, inf, dcur)
        return jnp.where(lane_out == k, val, acc)

    out_ref[...] = jax.lax.fori_loop(
        0, N_ANALOGS, step, jnp.zeros((N_CUR, N_ANALOGS), jnp.float32))


@functools.partial(jax.jit, static_argnames=())
def kernel(x_t, x_h, y_h, feature_weights):
    xh_t = jnp.pad(x_h.T, ((0, 0), (0, HPAD - H)))
    y_al = jnp.pad(y_h[TW:H - TW, 0], (0, H - N_HIST))[None, :]
    fw = feature_weights[:, None]

    out = pl.pallas_call(
        _anen_body,
        out_shape=jax.ShapeDtypeStruct((N_CUR, N_ANALOGS), jnp.float32),
        scratch_shapes=[pltpu.VMEM((N_CUR, H), jnp.float32)],
    )(x_t, xh_t, y_al, fw)
    return out
